# Initial kernel scaffold; baseline (speedup 1.0000x reference)
import jax
import jax.numpy as jnp
from jax.experimental import pallas as pl


def kernel(x, edge_index, W1, b1, W2, b2):
    raise NotImplementedError("write your pallas kernel here")


# speedup vs baseline: 2.3043x; 2.3043x over previous
import functools

import jax
import jax.numpy as jnp
from jax import lax
from jax.experimental import pallas as pl
from jax.experimental.pallas import tpu as pltpu
from jax.experimental.pallas import tpu_sc as plsc

N_NODES = 10000
N_EDGES = 320000
D = 128

NC = 2
NS = 16
NW = NC * NS
E_PER_W = N_EDGES // NW
CHUNK = 80
N_CHUNKS = E_PER_W // CHUNK
ROWS_PER_S = N_NODES // NS

@functools.cache
def _build_sc_kernels():
    mesh = plsc.VectorSubcoreMesh(core_axis_name="c", subcore_axis_name="s")

    deg_kernel = functools.partial(
        pl.kernel,
        mesh=mesh,
        out_type=[
            jax.ShapeDtypeStruct((NC, NS, ROWS_PER_S, 16), jnp.float32),
            jax.ShapeDtypeStruct((NC, NS, ROWS_PER_S, 16), jnp.float32),
        ],
        scratch_types=[
            pltpu.VMEM((2, CHUNK), jnp.int32),
            pltpu.VMEM((CHUNK, 16), jnp.float32),
            pltpu.VMEM((25, 16), jnp.float32),
            pltpu.VMEM_SHARED((N_NODES, 16), jnp.float32),
            pltpu.VMEM_SHARED((N_NODES, 16), jnp.float32),
        ],
    )(_deg_body)

    segsum_kernel = functools.partial(
        pl.kernel,
        mesh=mesh,
        out_type=jax.ShapeDtypeStruct((NC, NS, ROWS_PER_S, D), jnp.float32),
        scratch_types=[
            pltpu.VMEM((2, CHUNK), jnp.int32),
            pltpu.VMEM((CHUNK, D), jnp.float32),
            pltpu.VMEM_SHARED((N_NODES, D), jnp.float32),
            pltpu.SemaphoreType.DMA,
        ],
    )(_segsum_body)

    return deg_kernel, segsum_kernel


def _deg_body(src_hbm, dst_hbm, dego_hbm, degi_hbm,
              idx_v, ones_v, zbuf_v, dego_sh, degi_sh):
    c = lax.axis_index("c")
    s = lax.axis_index("s")
    wid = s * NC + c
    base = wid * E_PER_W

    one16 = jnp.ones((16,), jnp.float32)
    zero16 = jnp.zeros((16,), jnp.float32)
    for r in range(CHUNK):
        ones_v[r, :] = one16
    for r in range(25):
        zbuf_v[r, :] = zero16

    for j in range(ROWS_PER_S // 25):
        sl = pl.ds(s * ROWS_PER_S + j * 25, 25)
        pltpu.sync_copy(zbuf_v, dego_sh.at[sl])
        pltpu.sync_copy(zbuf_v, degi_sh.at[sl])
    plsc.subcore_barrier()

    def body(g, carry):
        off = base + g * CHUNK
        pltpu.sync_copy(src_hbm.at[pl.ds(off, CHUNK)], idx_v.at[0])
        pltpu.sync_copy(dst_hbm.at[pl.ds(off, CHUNK)], idx_v.at[1])
        pltpu.sync_copy(ones_v, dego_sh.at[idx_v.at[0]], add=True)
        pltpu.sync_copy(ones_v, degi_sh.at[idx_v.at[1]], add=True)
        return carry

    lax.fori_loop(0, N_CHUNKS, body, 0)
    plsc.subcore_barrier()

    sl = pl.ds(s * ROWS_PER_S, ROWS_PER_S)
    pltpu.sync_copy(dego_sh.at[sl], dego_hbm.at[c, s])
    pltpu.sync_copy(degi_sh.at[sl], degi_hbm.at[c, s])


def _segsum_body(h_hbm, src_hbm, dst_hbm, zer_hbm, out_hbm,
                 idx_v, rows_v, agg_sh, sem):
    c = lax.axis_index("c")
    s = lax.axis_index("s")
    wid = s * NC + c
    base = wid * E_PER_W

    for j in range(5):
        pltpu.sync_copy(
            zer_hbm, agg_sh.at[pl.ds(s * ROWS_PER_S + j * 125, 125)])
    plsc.subcore_barrier()

    def body(g, carry):
        off = base + g * CHUNK
        pltpu.sync_copy(src_hbm.at[pl.ds(off, CHUNK)], idx_v.at[0])
        pltpu.sync_copy(dst_hbm.at[pl.ds(off, CHUNK)], idx_v.at[1])
        pltpu.async_copy(h_hbm.at[idx_v.at[0]], rows_v, sem).wait()
        pltpu.sync_copy(rows_v, agg_sh.at[idx_v.at[1]], add=True)
        return carry

    lax.fori_loop(0, N_CHUNKS, body, 0)
    plsc.subcore_barrier()

    sl = pl.ds(s * ROWS_PER_S, ROWS_PER_S)
    pltpu.sync_copy(agg_sh.at[sl], out_hbm.at[c, s])


def _norm_col(ref):
    deg = ref[0, :, 0:1] + ref[1, :, 0:1]
    return lax.rsqrt(jnp.maximum(deg, 1.0))


def _mm1_body(x_ref, w1_ref, dego_ref, o_ref):
    no = _norm_col(dego_ref)
    o_ref[...] = jnp.dot(x_ref[...] * no, w1_ref[...],
                         preferred_element_type=jnp.float32)


def _mm2_body(a_ref, dego_ref, degi_ref, b1_ref, w2_ref, o_ref):
    a = a_ref[0] + a_ref[1]
    ni = _norm_col(degi_ref)
    no = _norm_col(dego_ref)
    h1 = jnp.maximum(a * ni + b1_ref[...], 0.0)
    o_ref[...] = jnp.dot(h1 * no, w2_ref[...],
                         preferred_element_type=jnp.float32)


def _final_body(a_ref, degi_ref, b2_ref, o_ref):
    a = a_ref[0] + a_ref[1]
    ni = _norm_col(degi_ref)
    o_ref[...] = a * ni + b2_ref[...]


_BLK = 1000
_GRID = N_NODES // _BLK

_spec_nd = pl.BlockSpec((_BLK, D), lambda i: (i, 0))
_spec_w = pl.BlockSpec((D, D), lambda i: (0, 0))
_spec_b = pl.BlockSpec((1, D), lambda i: (0, 0))
_spec_agg = pl.BlockSpec((NC, _BLK, D), lambda i: (0, i, 0))
_spec_deg = _spec_agg
_out_nd = jax.ShapeDtypeStruct((N_NODES, D), jnp.float32)


def kernel(x, edge_index, W1, b1, W2, b2):
    zer128 = jnp.zeros((125, D), jnp.float32)
    ones_mat = jnp.ones((N_NODES, D), jnp.float32)
    edge_index = edge_index.astype(jnp.int32)
    src = edge_index[0]
    dst = edge_index[1]
    deg_kernel, segsum_kernel = _build_sc_kernels()

    degi = segsum_kernel(ones_mat, src, dst, zer128).reshape(NC, N_NODES, D)
    dego = segsum_kernel(ones_mat, dst, src, zer128).reshape(NC, N_NODES, D)

    h0 = pl.pallas_call(
        _mm1_body,
        grid=(_GRID,),
        in_specs=[_spec_nd, _spec_w, _spec_deg],
        out_specs=_spec_nd,
        out_shape=_out_nd,
    )(x, W1, dego)

    agg1 = segsum_kernel(h0, src, dst, zer128).reshape(NC, N_NODES, D)

    h2 = pl.pallas_call(
        _mm2_body,
        grid=(_GRID,),
        in_specs=[_spec_agg, _spec_deg, _spec_deg, _spec_b, _spec_w],
        out_specs=_spec_nd,
        out_shape=_out_nd,
    )(agg1, dego, degi, b1.reshape(1, D), W2)

    agg2 = segsum_kernel(h2, src, dst, zer128).reshape(NC, N_NODES, D)

    out = pl.pallas_call(
        _final_body,
        grid=(_GRID,),
        in_specs=[_spec_agg, _spec_deg, _spec_b],
        out_specs=_spec_nd,
        out_shape=_out_nd,
    )(agg2, degi, b2.reshape(1, D))

    return out


# speedup vs baseline: 3.7241x; 1.6162x over previous
import functools

import jax
import jax.numpy as jnp
from jax import lax
from jax.experimental import pallas as pl
from jax.experimental.pallas import tpu as pltpu
from jax.experimental.pallas import tpu_sc as plsc

N_NODES = 10000
N_EDGES = 320000
D = 128

NC = 2
NS = 16
NW = NC * NS
E_PER_W = N_EDGES // NW
CHUNK = 80
N_CHUNKS = E_PER_W // CHUNK
ROWS_PER_S = N_NODES // NS

@functools.cache
def _build_sc_kernels():
    mesh = plsc.VectorSubcoreMesh(core_axis_name="c", subcore_axis_name="s")

    deg_kernel = functools.partial(
        pl.kernel,
        mesh=mesh,
        compiler_params=pltpu.CompilerParams(needs_layout_passes=False),
        out_type=[
            jax.ShapeDtypeStruct((NC, NS, N_NODES), jnp.float32),
            jax.ShapeDtypeStruct((NC, NS, N_NODES), jnp.float32),
        ],
        scratch_types=[
            pltpu.VMEM((2, CHUNK), jnp.int32),
            pltpu.VMEM((N_NODES,), jnp.float32),
            pltpu.VMEM((N_NODES,), jnp.float32),
        ],
    )(_deg_body)

    segsum_kernel = functools.partial(
        pl.kernel,
        mesh=mesh,
        out_type=jax.ShapeDtypeStruct((NC, NS, ROWS_PER_S, D), jnp.float32),
        scratch_types=[
            pltpu.VMEM((2, CHUNK), jnp.int32),
            pltpu.VMEM((CHUNK, D), jnp.float32),
            pltpu.VMEM_SHARED((N_NODES, D), jnp.float32),
            pltpu.SemaphoreType.DMA,
        ],
    )(_segsum_body)

    return deg_kernel, segsum_kernel


def _deg_body(src_hbm, dst_hbm, zer_hbm, dego_hbm, degi_hbm,
              idx_v, ho_v, hi_v):
    c = lax.axis_index("c")
    s = lax.axis_index("s")
    wid = s * NC + c
    base = wid * E_PER_W

    pltpu.sync_copy(zer_hbm, ho_v)
    pltpu.sync_copy(zer_hbm, hi_v)

    one16 = jnp.ones((16,), jnp.float32)

    def body(g, carry):
        off = base + g * CHUNK
        pltpu.sync_copy(src_hbm.at[pl.ds(off, CHUNK)], idx_v.at[0])
        pltpu.sync_copy(dst_hbm.at[pl.ds(off, CHUNK)], idx_v.at[1])
        for j in range(CHUNK // 16):
            iv_s = idx_v[0, pl.ds(j * 16, 16)]
            iv_d = idx_v[1, pl.ds(j * 16, 16)]
            plsc.addupdate_scatter(ho_v, [iv_s], one16)
            plsc.addupdate_scatter(hi_v, [iv_d], one16)
        return carry

    lax.fori_loop(0, N_CHUNKS, body, 0)

    pltpu.sync_copy(ho_v, dego_hbm.at[c, s])
    pltpu.sync_copy(hi_v, degi_hbm.at[c, s])


def _segsum_body(h_hbm, src_hbm, dst_hbm, zer_hbm, out_hbm,
                 idx_v, rows_v, agg_sh, sem):
    c = lax.axis_index("c")
    s = lax.axis_index("s")
    wid = s * NC + c
    base = wid * E_PER_W

    for j in range(5):
        pltpu.sync_copy(
            zer_hbm, agg_sh.at[pl.ds(s * ROWS_PER_S + j * 125, 125)])
    plsc.subcore_barrier()

    def body(g, carry):
        off = base + g * CHUNK
        pltpu.sync_copy(src_hbm.at[pl.ds(off, CHUNK)], idx_v.at[0])
        pltpu.sync_copy(dst_hbm.at[pl.ds(off, CHUNK)], idx_v.at[1])
        pltpu.async_copy(h_hbm.at[idx_v.at[0]], rows_v, sem).wait()
        pltpu.sync_copy(rows_v, agg_sh.at[idx_v.at[1]], add=True)
        return carry

    lax.fori_loop(0, N_CHUNKS, body, 0)
    plsc.subcore_barrier()

    sl = pl.ds(s * ROWS_PER_S, ROWS_PER_S)
    pltpu.sync_copy(agg_sh.at[sl], out_hbm.at[c, s])


def _norms_body(ho_ref, hi_ref, no_ref, ni_ref):
    dego = jnp.sum(ho_ref[...], axis=(0, 1))[:, None]
    degi = jnp.sum(hi_ref[...], axis=(0, 1))[:, None]
    no_ref[...] = lax.rsqrt(jnp.maximum(dego, 1.0))
    ni_ref[...] = lax.rsqrt(jnp.maximum(degi, 1.0))


def _mm1_body(x_ref, w1_ref, no_ref, o_ref):
    o_ref[...] = jnp.dot(x_ref[...] * no_ref[...], w1_ref[...],
                         preferred_element_type=jnp.float32)


def _mm2_body(a_ref, no_ref, ni_ref, b1_ref, w2_ref, o_ref):
    a = a_ref[0] + a_ref[1]
    h1 = jnp.maximum(a * ni_ref[...] + b1_ref[...], 0.0)
    o_ref[...] = jnp.dot(h1 * no_ref[...], w2_ref[...],
                         preferred_element_type=jnp.float32)


def _final_body(a_ref, ni_ref, b2_ref, o_ref):
    a = a_ref[0] + a_ref[1]
    o_ref[...] = a * ni_ref[...] + b2_ref[...]


_BLK = 1000
_GRID = N_NODES // _BLK

_spec_nd = pl.BlockSpec((_BLK, D), lambda i: (i, 0))
_spec_w = pl.BlockSpec((D, D), lambda i: (0, 0))
_spec_b = pl.BlockSpec((1, D), lambda i: (0, 0))
_spec_agg = pl.BlockSpec((NC, _BLK, D), lambda i: (0, i, 0))
_spec_norm = pl.BlockSpec((_BLK, 1), lambda i: (i, 0))
_out_nd = jax.ShapeDtypeStruct((N_NODES, D), jnp.float32)


def kernel(x, edge_index, W1, b1, W2, b2):
    zer128 = jnp.zeros((125, D), jnp.float32)
    zer1d = jnp.zeros((N_NODES,), jnp.float32)
    edge_index = edge_index.astype(jnp.int32)
    src = edge_index[0]
    dst = edge_index[1]
    deg_kernel, segsum_kernel = _build_sc_kernels()

    dego, degi = deg_kernel(src, dst, zer1d)

    no, ni = pl.pallas_call(
        _norms_body,
        grid=(1,),
        in_specs=[pl.BlockSpec((NC, NS, N_NODES), lambda i: (0, 0, 0))] * 2,
        out_specs=[pl.BlockSpec((N_NODES, 1), lambda i: (0, 0))] * 2,
        out_shape=[jax.ShapeDtypeStruct((N_NODES, 1), jnp.float32)] * 2,
    )(dego, degi)

    h0 = pl.pallas_call(
        _mm1_body,
        grid=(_GRID,),
        in_specs=[_spec_nd, _spec_w, _spec_norm],
        out_specs=_spec_nd,
        out_shape=_out_nd,
    )(x, W1, no)

    agg1 = segsum_kernel(h0, src, dst, zer128).reshape(NC, N_NODES, D)

    h2 = pl.pallas_call(
        _mm2_body,
        grid=(_GRID,),
        in_specs=[_spec_agg, _spec_norm, _spec_norm, _spec_b, _spec_w],
        out_specs=_spec_nd,
        out_shape=_out_nd,
    )(agg1, no, ni, b1.reshape(1, D), W2)

    agg2 = segsum_kernel(h2, src, dst, zer128).reshape(NC, N_NODES, D)

    out = pl.pallas_call(
        _final_body,
        grid=(_GRID,),
        in_specs=[_spec_agg, _spec_norm, _spec_b],
        out_specs=_spec_nd,
        out_shape=_out_nd,
    )(agg2, ni, b2.reshape(1, D))

    return out


# speedup vs baseline: 6.4899x; 1.7427x over previous
import functools

import jax
import jax.numpy as jnp
from jax import lax
from jax.experimental import pallas as pl
from jax.experimental.pallas import tpu as pltpu
from jax.experimental.pallas import tpu_sc as plsc

N_NODES = 10000
N_EDGES = 320000
D = 128

NC = 2
NS = 16
NW = NC * NS
E_PER_W = N_EDGES // NW
CHUNK = 40
N_CHUNKS = E_PER_W // CHUNK
NBUF = 5
ROWS_PER_S = N_NODES // NS

@functools.cache
def _build_sc_kernels():
    mesh = plsc.VectorSubcoreMesh(core_axis_name="c", subcore_axis_name="s")

    deg_kernel = functools.partial(
        pl.kernel,
        mesh=mesh,
        compiler_params=pltpu.CompilerParams(needs_layout_passes=False),
        out_type=[
            jax.ShapeDtypeStruct((NC, NS, N_NODES), jnp.float32),
            jax.ShapeDtypeStruct((NC, NS, N_NODES), jnp.float32),
        ],
        scratch_types=[
            pltpu.VMEM((E_PER_W,), jnp.int32),
            pltpu.VMEM((E_PER_W,), jnp.int32),
            pltpu.VMEM((N_NODES,), jnp.float32),
            pltpu.VMEM((N_NODES,), jnp.float32),
        ],
    )(_deg_body)

    segsum_kernel = functools.partial(
        pl.kernel,
        mesh=mesh,
        out_type=jax.ShapeDtypeStruct((NC, NS, ROWS_PER_S, D), jnp.float32),
        scratch_types=[
            pltpu.VMEM((NBUF, 2, CHUNK), jnp.int32),
            pltpu.VMEM((NBUF, CHUNK, D), jnp.float32),
            pltpu.VMEM_SHARED((N_NODES, D), jnp.float32),
        ] + [pltpu.SemaphoreType.DMA] * (3 * NBUF),
    )(_segsum_body)

    return deg_kernel, segsum_kernel


def _deg_body(src_hbm, dst_hbm, zer_hbm, dego_hbm, degi_hbm,
              sidx_v, didx_v, ho_v, hi_v):
    c = lax.axis_index("c")
    s = lax.axis_index("s")
    wid = s * NC + c
    base = wid * E_PER_W

    pltpu.sync_copy(src_hbm.at[pl.ds(base, E_PER_W)], sidx_v)
    pltpu.sync_copy(dst_hbm.at[pl.ds(base, E_PER_W)], didx_v)
    pltpu.sync_copy(zer_hbm, ho_v)
    pltpu.sync_copy(zer_hbm, hi_v)

    one16 = jnp.ones((16,), jnp.float32)

    def body(g, carry):
        off = g * 16
        plsc.addupdate_scatter(ho_v, [sidx_v[pl.ds(off, 16)]], one16)
        plsc.addupdate_scatter(hi_v, [didx_v[pl.ds(off, 16)]], one16)
        return carry

    lax.fori_loop(0, E_PER_W // 16, body, 0)

    pltpu.sync_copy(ho_v, dego_hbm.at[c, s])
    pltpu.sync_copy(hi_v, degi_hbm.at[c, s])


def _segsum_body(h_hbm, srcg_hbm, dstg_hbm, zer_hbm, out_hbm,
                 idx_v, rows_v, agg_sh, *sems):
    c = lax.axis_index("c")
    s = lax.axis_index("s")
    wid = s * NC + c

    for j in range(5):
        pltpu.sync_copy(
            zer_hbm, agg_sh.at[pl.ds(s * ROWS_PER_S + j * 125, 125)])
    plsc.subcore_barrier()

    def body(k, carry):
        ips, dps, cps = [], [], []
        for u in range(NBUF):
            g = k * NBUF + u
            ips.append(pltpu.async_copy(
                srcg_hbm.at[wid, g], idx_v.at[u, 0], sems[NBUF + u]))
            dps.append(pltpu.async_copy(
                dstg_hbm.at[wid, g], idx_v.at[u, 1], sems[2 * NBUF + u]))
        for u in range(NBUF):
            ips[u].wait()
            cps.append(pltpu.async_copy(
                h_hbm.at[idx_v.at[u, 0]], rows_v.at[u], sems[u]))
        for u in range(NBUF):
            dps[u].wait()
            cps[u].wait()
            pltpu.sync_copy(rows_v.at[u], agg_sh.at[idx_v.at[u, 1]], add=True)
        return carry

    lax.fori_loop(0, N_CHUNKS // NBUF, body, 0)
    plsc.subcore_barrier()

    sl = pl.ds(s * ROWS_PER_S, ROWS_PER_S)
    pltpu.sync_copy(agg_sh.at[sl], out_hbm.at[c, s])


def _norms_body(ho_ref, hi_ref, no_ref, ni_ref):
    dego = jnp.sum(ho_ref[...], axis=(0, 1))[:, None]
    degi = jnp.sum(hi_ref[...], axis=(0, 1))[:, None]
    no_ref[...] = lax.rsqrt(jnp.maximum(dego, 1.0))
    ni_ref[...] = lax.rsqrt(jnp.maximum(degi, 1.0))


def _mm1_body(x_ref, w1_ref, no_ref, o_ref):
    o_ref[...] = jnp.dot(x_ref[...] * no_ref[...], w1_ref[...],
                         preferred_element_type=jnp.float32)


def _mm2_body(a_ref, no_ref, ni_ref, b1_ref, w2_ref, o_ref):
    a = a_ref[0] + a_ref[1]
    h1 = jnp.maximum(a * ni_ref[...] + b1_ref[...], 0.0)
    o_ref[...] = jnp.dot(h1 * no_ref[...], w2_ref[...],
                         preferred_element_type=jnp.float32)


def _final_body(a_ref, ni_ref, b2_ref, o_ref):
    a = a_ref[0] + a_ref[1]
    o_ref[...] = a * ni_ref[...] + b2_ref[...]


_BLK = 1000
_GRID = N_NODES // _BLK

_spec_nd = pl.BlockSpec((_BLK, D), lambda i: (i, 0))
_spec_w = pl.BlockSpec((D, D), lambda i: (0, 0))
_spec_b = pl.BlockSpec((1, D), lambda i: (0, 0))
_spec_agg = pl.BlockSpec((NC, _BLK, D), lambda i: (0, i, 0))
_spec_norm = pl.BlockSpec((_BLK, 1), lambda i: (i, 0))
_out_nd = jax.ShapeDtypeStruct((N_NODES, D), jnp.float32)


def kernel(x, edge_index, W1, b1, W2, b2):
    zer128 = jnp.zeros((125, D), jnp.float32)
    zer1d = jnp.zeros((N_NODES,), jnp.float32)
    edge_index = edge_index.astype(jnp.int32)
    src = edge_index[0]
    dst = edge_index[1]
    deg_kernel, segsum_kernel = _build_sc_kernels()

    srcg = src.reshape(NW, N_CHUNKS, CHUNK)
    dstg = dst.reshape(NW, N_CHUNKS, CHUNK)

    dego, degi = deg_kernel(src, dst, zer1d)

    no, ni = pl.pallas_call(
        _norms_body,
        grid=(1,),
        in_specs=[pl.BlockSpec((NC, NS, N_NODES), lambda i: (0, 0, 0))] * 2,
        out_specs=[pl.BlockSpec((N_NODES, 1), lambda i: (0, 0))] * 2,
        out_shape=[jax.ShapeDtypeStruct((N_NODES, 1), jnp.float32)] * 2,
    )(dego, degi)

    h0 = pl.pallas_call(
        _mm1_body,
        grid=(_GRID,),
        in_specs=[_spec_nd, _spec_w, _spec_norm],
        out_specs=_spec_nd,
        out_shape=_out_nd,
    )(x, W1, no)

    agg1 = segsum_kernel(h0, srcg, dstg, zer128).reshape(NC, N_NODES, D)

    h2 = pl.pallas_call(
        _mm2_body,
        grid=(_GRID,),
        in_specs=[_spec_agg, _spec_norm, _spec_norm, _spec_b, _spec_w],
        out_specs=_spec_nd,
        out_shape=_out_nd,
    )(agg1, no, ni, b1.reshape(1, D), W2)

    agg2 = segsum_kernel(h2, srcg, dstg, zer128).reshape(NC, N_NODES, D)

    out = pl.pallas_call(
        _final_body,
        grid=(_GRID,),
        in_specs=[_spec_agg, _spec_norm, _spec_b],
        out_specs=_spec_nd,
        out_shape=_out_nd,
    )(agg2, ni, b2.reshape(1, D))

    return out
